# Initial kernel scaffold; baseline (speedup 1.0000x reference)
import functools

import jax
import jax.numpy as jnp
from jax import lax
from jax.experimental import pallas as pl
from jax.experimental.pallas import tpu as pltpu
from jax.experimental.pallas import tpu_sc as plsc


def _pre_stage(x, W, ALR):
    N = x.shape[0]
    Dh = W.shape[1]

    def body(x_ref, w_ref, a_ref, feat_ref, elr_ref):
        feat = jnp.dot(x_ref[...], w_ref[...], preferred_element_type=jnp.float32)
        feat_ref[...] = feat
        elr_ref[...] = jnp.dot(feat, a_ref[...], preferred_element_type=jnp.float32)

    return pl.pallas_call(
        body,
        out_shape=[
            jax.ShapeDtypeStruct((N, Dh), jnp.float32),
            jax.ShapeDtypeStruct((N, 128), jnp.float32),
        ],
    )(x, W, ALR)


def _bn(h, gamma, beta, eps=1e-5):
    m = jnp.mean(h, axis=0, keepdims=True)
    v = jnp.mean((h - m) ** 2, axis=0, keepdims=True)
    return (h - m) / jnp.sqrt(v + eps) * gamma + beta


def _mid_stage(acc, b1, gamma, beta, W2, ALR2):
    N = acc.shape[1]
    D = b1.shape[1]
    Do = W2.shape[1]

    def body(acc_ref, b_ref, g_ref, be_ref, w_ref, a_ref, feat_ref, elr_ref):
        a = acc_ref[0] + acc_ref[1]
        denom = a[:, D:D + 1]
        denom = jnp.where(denom == 0.0, 1.0, denom)
        rst = a[:, :D] / denom + b_ref[...]
        h = jnp.where(rst > 0, rst, jnp.exp(jnp.minimum(rst, 0.0)) - 1.0)
        h = _bn(h, g_ref[...], be_ref[...])
        h = _bn(h, g_ref[...], be_ref[...])
        feat = jnp.dot(h, w_ref[...], preferred_element_type=jnp.float32)
        feat_ref[...] = feat
        elr_ref[...] = jnp.dot(feat, a_ref[...], preferred_element_type=jnp.float32)

    return pl.pallas_call(
        body,
        out_shape=[
            jax.ShapeDtypeStruct((N, Do), jnp.float32),
            jax.ShapeDtypeStruct((N, 128), jnp.float32),
        ],
    )(acc, b1, gamma, beta, W2, ALR2)


def _final_stage(acc, b2):
    N = acc.shape[1]
    D = b2.shape[1]

    def body(acc_ref, b_ref, out_ref):
        a = acc_ref[0] + acc_ref[1]
        denom = a[:, D:D + 1]
        denom = jnp.where(denom == 0.0, 1.0, denom)
        out_ref[...] = a[:, :D] / denom + b_ref[...]

    return pl.pallas_call(
        body,
        out_shape=jax.ShapeDtypeStruct((N, D), jnp.float32),
    )(acc, b2)


@functools.cache
def _make_edge_kernel(N, E, D, K):
    Dp = D + 16
    info = plsc.get_sparse_core_info()
    NC, NS = info.num_cores, info.num_subcores
    NW = NC * NS
    EPT = E // NW
    assert E % NW == 0 and EPT % K == 0 and K % 16 == 0
    NCHUNK = EPT // K
    RPT = N // NS
    assert N % NS == 0
    ZR = 125
    assert RPT % ZR == 0
    mesh = plsc.VectorSubcoreMesh(core_axis_name="c", subcore_axis_name="s")

    @functools.partial(
        pl.kernel,
        mesh=mesh,
        out_type=jax.ShapeDtypeStruct((NC, N, Dp), jnp.float32),
        scratch_types=[
            pltpu.VMEM((K,), jnp.int32),
            pltpu.VMEM((K,), jnp.int32),
            pltpu.VMEM((K, D), jnp.float32),
            pltpu.VMEM((K,), jnp.float32),
            pltpu.VMEM((K,), jnp.float32),
            pltpu.VMEM((K,), jnp.float32),
            pltpu.VMEM((K, Dp), jnp.float32),
            pltpu.VMEM((ZR, Dp), jnp.float32),
            pltpu.VMEM_SHARED((N, Dp), jnp.float32),
        ],
    )
    def edge_kernel(src_h, dst_h, feat_h, el_h, er_h, out_h,
                    sidx, didx, rows, els, erd, wbuf, scb, zb, acc):
        cid = lax.axis_index("c")
        sid = lax.axis_index("s")
        tid = cid * NS + sid
        zeros16 = jnp.zeros((16,), jnp.float32)

        def zrow(i, carry):
            for j in range(Dp // 16):
                zb[i, pl.ds(j * 16, 16)] = zeros16
            return carry

        lax.fori_loop(0, ZR, zrow, 0)

        def zpad(i, carry):
            scb[i, pl.ds(D, 16)] = zeros16
            return carry

        lax.fori_loop(0, K, zpad, 0)

        for q in range(RPT // ZR):
            pltpu.sync_copy(zb, acc.at[pl.ds(sid * RPT + q * ZR, ZR)])
        plsc.subcore_barrier()

        lane = lax.iota(jnp.int32, 16)
        colD = jnp.full((16,), D, jnp.int32)

        def chunk(ci, carry):
            b = tid * EPT + ci * K
            pltpu.sync_copy(src_h.at[pl.ds(b, K)], sidx)
            pltpu.sync_copy(dst_h.at[pl.ds(b, K)], didx)
            pltpu.sync_copy(feat_h.at[sidx], rows)
            pltpu.sync_copy(el_h.at[sidx], els)
            pltpu.sync_copy(er_h.at[didx], erd)
            for g in range(K // 16):
                e = els[pl.ds(g * 16, 16)] + erd[pl.ds(g * 16, 16)]
                e = jnp.where(e >= 0.0, e, e * 0.2)
                w = jnp.exp(e)
                wbuf[pl.ds(g * 16, 16)] = w
                plsc.store_scatter(scb, [g * 16 + lane, colD], w)

            def edge(k, c2):
                wv = plsc.load_gather(wbuf, [jnp.full((16,), k, jnp.int32)])
                for j in range(D // 16):
                    scb[k, pl.ds(j * 16, 16)] = rows[k, pl.ds(j * 16, 16)] * wv
                return c2

            lax.fori_loop(0, K, edge, 0)
            pltpu.sync_copy(scb, acc.at[didx], add=True)
            return carry

        lax.fori_loop(0, NCHUNK, chunk, 0)
        plsc.subcore_barrier()

        pltpu.sync_copy(acc.at[pl.ds(sid * RPT, RPT)],
                        out_h.at[cid, pl.ds(sid * RPT, RPT)])

    return edge_kernel


def kernel(x, edge_index, W1, al1, ar1, b1, gamma, beta, W2, al2, ar2, b2):
    N, Din = x.shape
    E = edge_index.shape[1]
    Dh = W1.shape[1]
    Do = W2.shape[1]

    src = edge_index[0].astype(jnp.int32)
    dst = edge_index[1].astype(jnp.int32)

    ALR1 = jnp.concatenate(
        [al1[0][:, None], ar1[0][:, None], jnp.zeros((Dh, 126), jnp.float32)], axis=1)
    ALR2 = jnp.concatenate(
        [al2[0][:, None], ar2[0][:, None], jnp.zeros((Do, 126), jnp.float32)], axis=1)

    feat1, elr1 = _pre_stage(x, W1, ALR1)
    el1 = elr1[:, 0]
    er1 = elr1[:, 1]

    acc1 = _make_edge_kernel(N, E, Dh, 80)(src, dst, feat1, el1, er1)

    feat2, elr2 = _mid_stage(acc1, b1.reshape(1, Dh), gamma.reshape(1, Dh),
                             beta.reshape(1, Dh), W2, ALR2)
    el2 = elr2[:, 0]
    er2 = elr2[:, 1]

    acc2 = _make_edge_kernel(N, E, Do, 80)(src, dst, feat2, el2, er2)

    return _final_stage(acc2, b2.reshape(1, Do))


# speedup vs baseline: 12.8936x; 12.8936x over previous
import functools

import jax
import jax.numpy as jnp
from jax import lax
from jax.experimental import pallas as pl
from jax.experimental.pallas import tpu as pltpu
from jax.experimental.pallas import tpu_sc as plsc


def _pre_stage(x, W, ALR):
    N = x.shape[0]
    Dh = W.shape[1]

    def body(x_ref, w_ref, a_ref, feat_ref, elr_ref):
        feat = jnp.dot(x_ref[...], w_ref[...], preferred_element_type=jnp.float32)
        feat_ref[...] = feat
        elr_ref[...] = jnp.dot(feat, a_ref[...], preferred_element_type=jnp.float32)

    return pl.pallas_call(
        body,
        out_shape=[
            jax.ShapeDtypeStruct((N, Dh), jnp.float32),
            jax.ShapeDtypeStruct((N, 128), jnp.float32),
        ],
    )(x, W, ALR)


def _bn(h, gamma, beta, eps=1e-5):
    m = jnp.mean(h, axis=0, keepdims=True)
    v = jnp.mean((h - m) ** 2, axis=0, keepdims=True)
    return (h - m) / jnp.sqrt(v + eps) * gamma + beta


def _mid_stage(acc, N, b1, gamma, beta, W2, ALR2):
    D = b1.shape[1]
    Do = W2.shape[1]

    def body(acc_ref, b_ref, g_ref, be_ref, w_ref, a_ref, feat_ref, elr_ref):
        a = acc_ref[0, :N] + acc_ref[1, :N]
        denom = a[:, D:D + 1]
        denom = jnp.where(denom == 0.0, 1.0, denom)
        rst = a[:, :D] / denom + b_ref[...]
        h = jnp.where(rst > 0, rst, jnp.exp(jnp.minimum(rst, 0.0)) - 1.0)
        h = _bn(h, g_ref[...], be_ref[...])
        h = _bn(h, g_ref[...], be_ref[...])
        feat = jnp.dot(h, w_ref[...], preferred_element_type=jnp.float32)
        feat_ref[...] = feat
        elr_ref[...] = jnp.dot(feat, a_ref[...], preferred_element_type=jnp.float32)

    return pl.pallas_call(
        body,
        out_shape=[
            jax.ShapeDtypeStruct((N, Do), jnp.float32),
            jax.ShapeDtypeStruct((N, 128), jnp.float32),
        ],
    )(acc, b1, gamma, beta, W2, ALR2)


def _final_stage(acc, N, b2):
    D = b2.shape[1]

    def body(acc_ref, b_ref, out_ref):
        a = acc_ref[0, :N] + acc_ref[1, :N]
        denom = a[:, D:D + 1]
        denom = jnp.where(denom == 0.0, 1.0, denom)
        out_ref[...] = a[:, :D] / denom + b_ref[...]

    return pl.pallas_call(
        body,
        out_shape=jax.ShapeDtypeStruct((N, D), jnp.float32),
    )(acc, b2)


@functools.cache
def _make_edge_kernel(N, Np, E, D, K):
    Dp = D + 16
    info = plsc.get_sparse_core_info()
    NC, NS = info.num_cores, info.num_subcores
    NW = NC * NS
    EPT = E // NW
    assert E % NW == 0 and EPT % K == 0 and K % 16 == 0
    NCHUNK = EPT // K
    RPT = Np // NS
    assert Np % NS == 0 and RPT % 8 == 0 and RPT % K == 0
    mesh = plsc.VectorSubcoreMesh(core_axis_name="c", subcore_axis_name="s")

    @functools.partial(
        pl.kernel,
        mesh=mesh,
        compiler_params=pltpu.CompilerParams(
            use_tc_tiling_on_sc=False, needs_layout_passes=False),
        out_type=jax.ShapeDtypeStruct((NC, Np, Dp), jnp.float32),
        scratch_types=[
            pltpu.VMEM((K,), jnp.int32),
            pltpu.VMEM((K,), jnp.int32),
            pltpu.VMEM((K, D), jnp.float32),
            pltpu.VMEM((K,), jnp.float32),
            pltpu.VMEM((K,), jnp.float32),
            pltpu.VMEM((K,), jnp.float32),
            pltpu.VMEM((K, Dp), jnp.float32),
            pltpu.VMEM_SHARED((Np, Dp), jnp.float32),
        ],
    )
    def edge_kernel(src_h, dst_h, feat_h, el_h, er_h, out_h,
                    sidx, didx, rows, els, erd, wbuf, scb, acc):
        cid = lax.axis_index("c")
        sid = lax.axis_index("s")
        tid = cid * NS + sid
        zeros16 = jnp.zeros((16,), jnp.float32)

        def zrow(i, carry):
            for j in range(Dp // 16):
                scb[i, pl.ds(j * 16, 16)] = zeros16
            return carry

        lax.fori_loop(0, K, zrow, 0)
        for q in range(RPT // K):
            pltpu.sync_copy(scb, acc.at[pl.ds(sid * RPT + q * K, K)])
        plsc.subcore_barrier()

        def chunk(ci, carry):
            b = tid * EPT + ci * K
            pltpu.sync_copy(src_h.at[pl.ds(b, K)], sidx)
            pltpu.sync_copy(dst_h.at[pl.ds(b, K)], didx)
            pltpu.sync_copy(feat_h.at[sidx], rows)
            pltpu.sync_copy(el_h.at[sidx], els)
            pltpu.sync_copy(er_h.at[didx], erd)
            for g in range(K // 16):
                e = els[pl.ds(g * 16, 16)] + erd[pl.ds(g * 16, 16)]
                e = jnp.where(e >= 0.0, e, e * 0.2)
                wbuf[pl.ds(g * 16, 16)] = jnp.exp(e)

            def edge(k, c2):
                wv = plsc.load_gather(wbuf, [jnp.full((16,), k, jnp.int32)])
                for j in range(D // 16):
                    scb[k, pl.ds(j * 16, 16)] = rows[k, pl.ds(j * 16, 16)] * wv
                scb[k, pl.ds(D, 16)] = wv
                return c2

            lax.fori_loop(0, K, edge, 0)
            pltpu.sync_copy(scb, acc.at[didx], add=True)
            return carry

        lax.fori_loop(0, NCHUNK, chunk, 0)
        plsc.subcore_barrier()

        pltpu.sync_copy(acc.at[pl.ds(sid * RPT, RPT)],
                        out_h.at[cid, pl.ds(sid * RPT, RPT)])

    return edge_kernel


def kernel(x, edge_index, W1, al1, ar1, b1, gamma, beta, W2, al2, ar2, b2):
    N, Din = x.shape
    E = edge_index.shape[1]
    Dh = W1.shape[1]
    Do = W2.shape[1]

    src = edge_index[0].astype(jnp.int32)
    dst = edge_index[1].astype(jnp.int32)

    ALR1 = jnp.concatenate(
        [al1[0][:, None], ar1[0][:, None], jnp.zeros((Dh, 126), jnp.float32)], axis=1)
    ALR2 = jnp.concatenate(
        [al2[0][:, None], ar2[0][:, None], jnp.zeros((Do, 126), jnp.float32)], axis=1)

    Np = ((N + 2047) // 2048) * 2048

    feat1, elr1 = _pre_stage(x, W1, ALR1)
    el1 = elr1[:, 0]
    er1 = elr1[:, 1]

    acc1 = _make_edge_kernel(N, Np, E, Dh, 80)(src, dst, feat1, el1, er1)

    feat2, elr2 = _mid_stage(acc1, N, b1.reshape(1, Dh), gamma.reshape(1, Dh),
                             beta.reshape(1, Dh), W2, ALR2)
    el2 = elr2[:, 0]
    er2 = elr2[:, 1]

    acc2 = _make_edge_kernel(N, Np, E, Do, 80)(src, dst, feat2, el2, er2)

    return _final_stage(acc2, N, b2.reshape(1, Do))


# speedup vs baseline: 43.2106x; 3.3513x over previous
import functools

import jax
import jax.numpy as jnp
from jax import lax
from jax.experimental import pallas as pl
from jax.experimental.pallas import tpu as pltpu
from jax.experimental.pallas import tpu_sc as plsc


def _pre_stage(x, W, ALR):
    N = x.shape[0]
    Dh = W.shape[1]

    def body(x_ref, w_ref, a_ref, feat_ref, elr_ref):
        feat = jnp.dot(x_ref[...], w_ref[...], preferred_element_type=jnp.float32)
        feat_ref[...] = feat
        elr_ref[...] = jnp.dot(feat, a_ref[...], preferred_element_type=jnp.float32)

    return pl.pallas_call(
        body,
        out_shape=[
            jax.ShapeDtypeStruct((N, Dh), jnp.float32),
            jax.ShapeDtypeStruct((N, 128), jnp.float32),
        ],
    )(x, W, ALR)


def _bn(h, gamma, beta, eps=1e-5):
    m = jnp.mean(h, axis=0, keepdims=True)
    v = jnp.mean((h - m) ** 2, axis=0, keepdims=True)
    return (h - m) / jnp.sqrt(v + eps) * gamma + beta


def _mid_stage(accf, accw, N, b1, gamma, beta, W2, ALR2):
    D = b1.shape[1]
    Do = W2.shape[1]

    def body(accf_ref, accw_ref, b_ref, g_ref, be_ref, w_ref, a_ref,
             feat_ref, elr_ref):
        a = accf_ref[0, :N] + accf_ref[1, :N]
        denom = accw_ref[0, :N, 0:1] + accw_ref[1, :N, 0:1]
        denom = jnp.where(denom == 0.0, 1.0, denom)
        rst = a / denom + b_ref[...]
        h = jnp.where(rst > 0, rst, jnp.exp(jnp.minimum(rst, 0.0)) - 1.0)
        h = _bn(h, g_ref[...], be_ref[...])
        h = _bn(h, g_ref[...], be_ref[...])
        feat = jnp.dot(h, w_ref[...], preferred_element_type=jnp.float32)
        feat_ref[...] = feat
        elr_ref[...] = jnp.dot(feat, a_ref[...], preferred_element_type=jnp.float32)

    return pl.pallas_call(
        body,
        out_shape=[
            jax.ShapeDtypeStruct((N, Do), jnp.float32),
            jax.ShapeDtypeStruct((N, 128), jnp.float32),
        ],
    )(accf, accw, b1, gamma, beta, W2, ALR2)


def _final_stage(accf, accw, N, b2):
    D = b2.shape[1]

    def body(accf_ref, accw_ref, b_ref, out_ref):
        a = accf_ref[0, :N] + accf_ref[1, :N]
        denom = accw_ref[0, :N, 0:1] + accw_ref[1, :N, 0:1]
        denom = jnp.where(denom == 0.0, 1.0, denom)
        out_ref[...] = a / denom + b_ref[...]

    return pl.pallas_call(
        body,
        out_shape=jax.ShapeDtypeStruct((N, D), jnp.float32),
    )(accf, accw, b2)


@functools.cache
def _make_edge_kernel(N, Np, E, D, K):
    NBUF = 3
    info = plsc.get_sparse_core_info()
    NC, NS = info.num_cores, info.num_subcores
    NW = NC * NS
    EPT = E // NW
    assert E % NW == 0 and EPT % K == 0 and K % 16 == 0
    NCHUNK = EPT // K
    RPT = Np // NS
    assert Np % NS == 0 and RPT % 8 == 0 and RPT % K == 0
    mesh = plsc.VectorSubcoreMesh(core_axis_name="c", subcore_axis_name="s")

    buf_types = []
    for _ in range(NBUF):
        buf_types += [
            pltpu.VMEM((K,), jnp.int32),
            pltpu.VMEM((K,), jnp.int32),
            pltpu.VMEM((K, D), jnp.float32),
            pltpu.VMEM((K,), jnp.float32),
            pltpu.VMEM((K,), jnp.float32),
            pltpu.VMEM((K, 16), jnp.float32),
            pltpu.SemaphoreType.DMA,
            pltpu.SemaphoreType.DMA,
        ]

    @functools.partial(
        pl.kernel,
        mesh=mesh,
        compiler_params=pltpu.CompilerParams(
            use_tc_tiling_on_sc=False, needs_layout_passes=False),
        out_type=(jax.ShapeDtypeStruct((NC, Np, D), jnp.float32),
                  jax.ShapeDtypeStruct((NC, Np, 16), jnp.float32)),
        scratch_types=buf_types + [
            pltpu.VMEM_SHARED((Np, D), jnp.float32),
            pltpu.VMEM_SHARED((Np, 16), jnp.float32),
        ],
    )
    def edge_kernel(src_h, dst_h, feat_h, el_h, er_h, outf_h, outw_h, *scr):
        bufs = [scr[i * 8:(i + 1) * 8] for i in range(NBUF)]
        accf, accw = scr[NBUF * 8], scr[NBUF * 8 + 1]
        cid = lax.axis_index("c")
        sid = lax.axis_index("s")
        tid = cid * NS + sid
        zeros16 = jnp.zeros((16,), jnp.float32)

        rows0, wbr0 = bufs[0][2], bufs[0][5]

        def zrow(i, carry):
            for j in range(D // 16):
                rows0[i, pl.ds(j * 16, 16)] = zeros16
            wbr0[i, pl.ds(0, 16)] = zeros16
            return carry

        lax.fori_loop(0, K, zrow, 0)
        for q in range(RPT // K):
            pltpu.sync_copy(rows0, accf.at[pl.ds(sid * RPT + q * K, K)])
            pltpu.sync_copy(wbr0, accw.at[pl.ds(sid * RPT + q * K, K)])
        plsc.subcore_barrier()

        def issue_gathers(b, ci):
            sidx, didx, rows, els, erd, wbr, gsem, ssem = bufs[b]
            base = tid * EPT + ci * K
            pltpu.async_copy(src_h.at[pl.ds(base, K)], sidx, gsem)
            pltpu.async_copy(dst_h.at[pl.ds(base, K)], didx, gsem)

        def wait_idx_and_fetch(b):
            sidx, didx, rows, els, erd, wbr, gsem, ssem = bufs[b]
            pltpu.make_async_copy(src_h.at[pl.ds(0, K)], sidx, gsem).wait()
            pltpu.make_async_copy(dst_h.at[pl.ds(0, K)], didx, gsem).wait()
            pltpu.async_copy(feat_h.at[sidx], rows, gsem)
            pltpu.async_copy(el_h.at[sidx], els, gsem)
            pltpu.async_copy(er_h.at[didx], erd, gsem)

        def wait_gathers(b):
            sidx, didx, rows, els, erd, wbr, gsem, ssem = bufs[b]
            pltpu.make_async_copy(feat_h.at[sidx], rows, gsem).wait()
            pltpu.make_async_copy(el_h.at[sidx], els, gsem).wait()
            pltpu.make_async_copy(er_h.at[didx], erd, gsem).wait()

        def compute(b):
            sidx, didx, rows, els, erd, wbr, gsem, ssem = bufs[b]
            for g in range(K // 16):
                e = els[pl.ds(g * 16, 16)] + erd[pl.ds(g * 16, 16)]
                e = jnp.where(e >= 0.0, e, e * 0.2)
                els[pl.ds(g * 16, 16)] = jnp.exp(e)

            def edge(k, c2):
                wv = plsc.load_gather(els, [jnp.full((16,), k, jnp.int32)])
                for j in range(D // 16):
                    rows[k, pl.ds(j * 16, 16)] = rows[k, pl.ds(j * 16, 16)] * wv
                wbr[k, pl.ds(0, 16)] = wv
                return c2

            lax.fori_loop(0, K, edge, 0, unroll=2)

        def issue_scatter(b):
            sidx, didx, rows, els, erd, wbr, gsem, ssem = bufs[b]
            pltpu.async_copy(rows, accf.at[didx], ssem, add=True)
            pltpu.async_copy(wbr, accw.at[didx], ssem, add=True)

        def wait_scatter(b):
            sidx, didx, rows, els, erd, wbr, gsem, ssem = bufs[b]
            pltpu.make_async_copy(rows, accf.at[didx], ssem).wait()
            pltpu.make_async_copy(wbr, accw.at[didx], ssem).wait()

        issue_gathers(0, 0)
        issue_gathers(1, 1)
        wait_idx_and_fetch(0)
        wait_idx_and_fetch(1)

        def body(ci, carry):
            for r in range(NBUF):
                @pl.when(ci % NBUF == r)
                def _():
                    wait_gathers(r)
                    compute(r)
                    issue_scatter(r)
                    nb = (r + 2) % NBUF

                    @pl.when(ci + 2 < NCHUNK)
                    def _():
                        @pl.when(ci >= 1)
                        def _():
                            wait_scatter(nb)

                        issue_gathers(nb, ci + 2)
                        wait_idx_and_fetch(nb)
            return carry

        assert NCHUNK >= NBUF
        lax.fori_loop(0, NCHUNK, body, 0)
        for r in range(NBUF):
            wait_scatter(r)
        plsc.subcore_barrier()

        pltpu.sync_copy(accf.at[pl.ds(sid * RPT, RPT)],
                        outf_h.at[cid, pl.ds(sid * RPT, RPT)])
        pltpu.sync_copy(accw.at[pl.ds(sid * RPT, RPT)],
                        outw_h.at[cid, pl.ds(sid * RPT, RPT)])

    return edge_kernel


def kernel(x, edge_index, W1, al1, ar1, b1, gamma, beta, W2, al2, ar2, b2):
    N, Din = x.shape
    E = edge_index.shape[1]
    Dh = W1.shape[1]
    Do = W2.shape[1]

    src = edge_index[0].astype(jnp.int32)
    dst = edge_index[1].astype(jnp.int32)

    ALR1 = jnp.concatenate(
        [al1[0][:, None], ar1[0][:, None], jnp.zeros((Dh, 126), jnp.float32)], axis=1)
    ALR2 = jnp.concatenate(
        [al2[0][:, None], ar2[0][:, None], jnp.zeros((Do, 126), jnp.float32)], axis=1)

    Np = ((N + 2047) // 2048) * 2048

    feat1, elr1 = _pre_stage(x, W1, ALR1)
    el1 = elr1[:, 0]
    er1 = elr1[:, 1]

    acc1f, acc1w = _make_edge_kernel(N, Np, E, Dh, 80)(src, dst, feat1, el1, er1)

    feat2, elr2 = _mid_stage(acc1f, acc1w, N, b1.reshape(1, Dh),
                             gamma.reshape(1, Dh), beta.reshape(1, Dh), W2, ALR2)
    el2 = elr2[:, 0]
    er2 = elr2[:, 1]

    acc2f, acc2w = _make_edge_kernel(N, Np, E, Do, 80)(src, dst, feat2, el2, er2)

    return _final_stage(acc2f, acc2w, N, b2.reshape(1, Do))


# speedup vs baseline: 51.3283x; 1.1879x over previous
import functools

import jax
import jax.numpy as jnp
from jax import lax
from jax.experimental import pallas as pl
from jax.experimental.pallas import tpu as pltpu
from jax.experimental.pallas import tpu_sc as plsc


def _pre_stage(x, W, ALR):
    N = x.shape[0]
    Dh = W.shape[1]

    def body(x_ref, w_ref, a_ref, feat_ref, elr_ref):
        feat = jnp.dot(x_ref[...], w_ref[...], preferred_element_type=jnp.float32)
        feat_ref[...] = feat
        elr_ref[...] = jnp.dot(feat, a_ref[...], preferred_element_type=jnp.float32)

    return pl.pallas_call(
        body,
        out_shape=[
            jax.ShapeDtypeStruct((N, Dh), jnp.float32),
            jax.ShapeDtypeStruct((N, 128), jnp.float32),
        ],
    )(x, W, ALR)


def _bn(h, gamma, beta, eps=1e-5):
    m = jnp.mean(h, axis=0, keepdims=True)
    v = jnp.mean((h - m) ** 2, axis=0, keepdims=True)
    return (h - m) / jnp.sqrt(v + eps) * gamma + beta


def _mid_stage(accf, accw, N, b1, gamma, beta, W2, ALR2):
    D = b1.shape[1]
    Do = W2.shape[1]

    def body(accf_ref, accw_ref, b_ref, g_ref, be_ref, w_ref, a_ref,
             feat_ref, elr_ref):
        a = accf_ref[0, :N] + accf_ref[1, :N]
        denom = accw_ref[0, :N, 0:1] + accw_ref[1, :N, 0:1]
        denom = jnp.where(denom == 0.0, 1.0, denom)
        rst = a / denom + b_ref[...]
        h = jnp.where(rst > 0, rst, jnp.exp(jnp.minimum(rst, 0.0)) - 1.0)
        h = _bn(h, g_ref[...], be_ref[...])
        h = _bn(h, g_ref[...], be_ref[...])
        feat = jnp.dot(h, w_ref[...], preferred_element_type=jnp.float32)
        feat_ref[...] = feat
        elr_ref[...] = jnp.dot(feat, a_ref[...], preferred_element_type=jnp.float32)

    return pl.pallas_call(
        body,
        out_shape=[
            jax.ShapeDtypeStruct((N, Do), jnp.float32),
            jax.ShapeDtypeStruct((N, 128), jnp.float32),
        ],
    )(accf, accw, b1, gamma, beta, W2, ALR2)


def _final_stage(accf, accw, N, b2):
    D = b2.shape[1]

    def body(accf_ref, accw_ref, b_ref, out_ref):
        a = accf_ref[0, :N] + accf_ref[1, :N]
        denom = accw_ref[0, :N, 0:1] + accw_ref[1, :N, 0:1]
        denom = jnp.where(denom == 0.0, 1.0, denom)
        out_ref[...] = a / denom + b_ref[...]

    return pl.pallas_call(
        body,
        out_shape=jax.ShapeDtypeStruct((N, D), jnp.float32),
    )(accf, accw, b2)


@functools.cache
def _make_edge_kernel(N, Np, E, D, K):
    NBUF = 3
    info = plsc.get_sparse_core_info()
    NC, NS = info.num_cores, info.num_subcores
    NW = NC * NS
    EPT = E // NW
    assert E % NW == 0 and EPT % K == 0 and K % 16 == 0
    NCHUNK = EPT // K
    RPT = Np // NS
    assert Np % NS == 0 and RPT % 8 == 0 and RPT % K == 0
    mesh = plsc.VectorSubcoreMesh(core_axis_name="c", subcore_axis_name="s")

    buf_types = []
    for _ in range(NBUF):
        buf_types += [
            pltpu.VMEM((K,), jnp.int32),
            pltpu.VMEM((K,), jnp.int32),
            pltpu.VMEM((K, D), jnp.float32),
            pltpu.VMEM((K,), jnp.float32),
            pltpu.VMEM((K,), jnp.float32),
            pltpu.VMEM((K, 16), jnp.float32),
            pltpu.SemaphoreType.DMA,
            pltpu.SemaphoreType.DMA,
        ]

    @functools.partial(
        pl.kernel,
        mesh=mesh,
        compiler_params=pltpu.CompilerParams(
            use_tc_tiling_on_sc=False, needs_layout_passes=False),
        out_type=(jax.ShapeDtypeStruct((NC, Np, D), jnp.float32),
                  jax.ShapeDtypeStruct((NC, Np, 16), jnp.float32)),
        scratch_types=buf_types + [
            pltpu.VMEM_SHARED((Np, D), jnp.float32),
            pltpu.VMEM_SHARED((Np, 16), jnp.float32),
        ],
    )
    def edge_kernel(src_h, dst_h, feat_h, el_h, er_h, outf_h, outw_h, *scr):
        bufs = [scr[i * 8:(i + 1) * 8] for i in range(NBUF)]
        accf, accw = scr[NBUF * 8], scr[NBUF * 8 + 1]
        cid = lax.axis_index("c")
        sid = lax.axis_index("s")
        tid = cid * NS + sid
        zeros16 = jnp.zeros((16,), jnp.float32)

        rows0, wbr0 = bufs[0][2], bufs[0][5]

        def zrow(i, carry):
            for j in range(D // 16):
                rows0[i, pl.ds(j * 16, 16)] = zeros16
            wbr0[i, pl.ds(0, 16)] = zeros16
            return carry

        lax.fori_loop(0, K, zrow, 0)
        for q in range(RPT // K):
            pltpu.sync_copy(rows0, accf.at[pl.ds(sid * RPT + q * K, K)])
            pltpu.sync_copy(wbr0, accw.at[pl.ds(sid * RPT + q * K, K)])
        plsc.subcore_barrier()

        def issue_gathers(b, ci):
            sidx, didx, rows, els, erd, wbr, gsem, ssem = bufs[b]
            base = tid * EPT + ci * K
            pltpu.async_copy(src_h.at[pl.ds(base, K)], sidx, gsem)
            pltpu.async_copy(dst_h.at[pl.ds(base, K)], didx, gsem)

        def wait_idx_and_fetch(b):
            sidx, didx, rows, els, erd, wbr, gsem, ssem = bufs[b]
            pltpu.make_async_copy(src_h.at[pl.ds(0, K)], sidx, gsem).wait()
            pltpu.make_async_copy(dst_h.at[pl.ds(0, K)], didx, gsem).wait()
            pltpu.async_copy(feat_h.at[sidx], rows, gsem)
            pltpu.async_copy(el_h.at[sidx], els, gsem)
            pltpu.async_copy(er_h.at[didx], erd, gsem)

        def wait_gathers(b):
            sidx, didx, rows, els, erd, wbr, gsem, ssem = bufs[b]
            pltpu.make_async_copy(feat_h.at[sidx], rows, gsem).wait()
            pltpu.make_async_copy(el_h.at[sidx], els, gsem).wait()
            pltpu.make_async_copy(er_h.at[didx], erd, gsem).wait()

        def compute(b):
            sidx, didx, rows, els, erd, wbr, gsem, ssem = bufs[b]
            for g in range(K // 16):
                e = els[pl.ds(g * 16, 16)] + erd[pl.ds(g * 16, 16)]
                e = jnp.where(e >= 0.0, e, e * 0.2)
                els[pl.ds(g * 16, 16)] = jnp.exp(e)

            @plsc.parallel_loop(0, K, 1, unroll=4)
            def edge(k):
                wv = plsc.load_gather(els, [jnp.full((16,), k, jnp.int32)])
                for j in range(D // 16):
                    rows[k, pl.ds(j * 16, 16)] = rows[k, pl.ds(j * 16, 16)] * wv
                wbr[k, pl.ds(0, 16)] = wv

        def issue_scatter(b):
            sidx, didx, rows, els, erd, wbr, gsem, ssem = bufs[b]
            pltpu.async_copy(rows, accf.at[didx], ssem, add=True)
            pltpu.async_copy(wbr, accw.at[didx], ssem, add=True)

        def wait_scatter(b):
            sidx, didx, rows, els, erd, wbr, gsem, ssem = bufs[b]
            pltpu.make_async_copy(rows, accf.at[didx], ssem).wait()
            pltpu.make_async_copy(wbr, accw.at[didx], ssem).wait()

        issue_gathers(0, 0)
        issue_gathers(1, 1)
        wait_idx_and_fetch(0)
        wait_idx_and_fetch(1)

        def body(ci, carry):
            for r in range(NBUF):
                @pl.when(ci % NBUF == r)
                def _():
                    wait_gathers(r)
                    compute(r)
                    issue_scatter(r)
                    nb = (r + 2) % NBUF

                    @pl.when(ci + 2 < NCHUNK)
                    def _():
                        @pl.when(ci >= 1)
                        def _():
                            wait_scatter(nb)

                        issue_gathers(nb, ci + 2)
                        wait_idx_and_fetch(nb)
            return carry

        assert NCHUNK >= NBUF
        lax.fori_loop(0, NCHUNK, body, 0)
        for r in range(NBUF):
            wait_scatter(r)
        plsc.subcore_barrier()

        pltpu.sync_copy(accf.at[pl.ds(sid * RPT, RPT)],
                        outf_h.at[cid, pl.ds(sid * RPT, RPT)])
        pltpu.sync_copy(accw.at[pl.ds(sid * RPT, RPT)],
                        outw_h.at[cid, pl.ds(sid * RPT, RPT)])

    return edge_kernel


def kernel(x, edge_index, W1, al1, ar1, b1, gamma, beta, W2, al2, ar2, b2):
    N, Din = x.shape
    E = edge_index.shape[1]
    Dh = W1.shape[1]
    Do = W2.shape[1]

    src = edge_index[0].astype(jnp.int32)
    dst = edge_index[1].astype(jnp.int32)

    ALR1 = jnp.concatenate(
        [al1[0][:, None], ar1[0][:, None], jnp.zeros((Dh, 126), jnp.float32)], axis=1)
    ALR2 = jnp.concatenate(
        [al2[0][:, None], ar2[0][:, None], jnp.zeros((Do, 126), jnp.float32)], axis=1)

    Np = ((N + 2047) // 2048) * 2048

    feat1, elr1 = _pre_stage(x, W1, ALR1)
    el1 = elr1[:, 0]
    er1 = elr1[:, 1]

    acc1f, acc1w = _make_edge_kernel(N, Np, E, Dh, 80)(src, dst, feat1, el1, er1)

    feat2, elr2 = _mid_stage(acc1f, acc1w, N, b1.reshape(1, Dh),
                             gamma.reshape(1, Dh), beta.reshape(1, Dh), W2, ALR2)
    el2 = elr2[:, 0]
    er2 = elr2[:, 1]

    acc2f, acc2w = _make_edge_kernel(N, Np, E, Do, 80)(src, dst, feat2, el2, er2)

    return _final_stage(acc2f, acc2w, N, b2.reshape(1, Do))


# speedup vs baseline: 51.6592x; 1.0064x over previous
import functools

import jax
import jax.numpy as jnp
from jax import lax
from jax.experimental import pallas as pl
from jax.experimental.pallas import tpu as pltpu
from jax.experimental.pallas import tpu_sc as plsc


def _pre_stage(x, W, al, ar):
    N = x.shape[0]
    Dh = W.shape[1]

    def body(x_ref, w_ref, al_ref, ar_ref, feat_ref, el_ref, er_ref):
        feat = jnp.dot(x_ref[...], w_ref[...], preferred_element_type=jnp.float32)
        feat_ref[...] = feat
        el_ref[...] = jnp.sum(feat * al_ref[...], axis=1)
        er_ref[...] = jnp.sum(feat * ar_ref[...], axis=1)

    return pl.pallas_call(
        body,
        out_shape=[
            jax.ShapeDtypeStruct((N, Dh), jnp.float32),
            jax.ShapeDtypeStruct((N,), jnp.float32),
            jax.ShapeDtypeStruct((N,), jnp.float32),
        ],
    )(x, W, al, ar)


def _bn(h, gamma, beta, eps=1e-5):
    m = jnp.mean(h, axis=0, keepdims=True)
    v = jnp.mean((h - m) ** 2, axis=0, keepdims=True)
    return (h - m) / jnp.sqrt(v + eps) * gamma + beta


def _mid_stage(accf, accw, N, b1, gamma, beta, W2, al2, ar2):
    D = b1.shape[1]
    Do = W2.shape[1]

    def body(accf_ref, accw_ref, b_ref, g_ref, be_ref, w_ref, al_ref, ar_ref,
             feat_ref, el_ref, er_ref):
        a = accf_ref[0, :N] + accf_ref[1, :N]
        denom = accw_ref[0, :N, 0:1] + accw_ref[1, :N, 0:1]
        denom = jnp.where(denom == 0.0, 1.0, denom)
        rst = a / denom + b_ref[...]
        h = jnp.where(rst > 0, rst, jnp.exp(jnp.minimum(rst, 0.0)) - 1.0)
        h = _bn(h, g_ref[...], be_ref[...])
        h = _bn(h, g_ref[...], be_ref[...])
        feat = jnp.dot(h, w_ref[...], preferred_element_type=jnp.float32)
        feat_ref[...] = feat
        el_ref[...] = jnp.sum(feat * al_ref[...], axis=1)
        er_ref[...] = jnp.sum(feat * ar_ref[...], axis=1)

    return pl.pallas_call(
        body,
        out_shape=[
            jax.ShapeDtypeStruct((N, Do), jnp.float32),
            jax.ShapeDtypeStruct((N,), jnp.float32),
            jax.ShapeDtypeStruct((N,), jnp.float32),
        ],
    )(accf, accw, b1, gamma, beta, W2, al2, ar2)


def _final_stage(accf, accw, N, b2):
    D = b2.shape[1]

    def body(accf_ref, accw_ref, b_ref, out_ref):
        a = accf_ref[0, :N] + accf_ref[1, :N]
        denom = accw_ref[0, :N, 0:1] + accw_ref[1, :N, 0:1]
        denom = jnp.where(denom == 0.0, 1.0, denom)
        out_ref[...] = a / denom + b_ref[...]

    return pl.pallas_call(
        body,
        out_shape=jax.ShapeDtypeStruct((N, D), jnp.float32),
    )(accf, accw, b2)


@functools.cache
def _make_edge_kernel(N, Np, E, D, K):
    NBUF = 3
    info = plsc.get_sparse_core_info()
    NC, NS = info.num_cores, info.num_subcores
    NW = NC * NS
    EPT = E // NW
    assert E % NW == 0 and EPT % K == 0 and K % 16 == 0
    NCHUNK = EPT // K
    RPT = Np // NS
    assert Np % NS == 0 and RPT % 8 == 0 and RPT % K == 0
    mesh = plsc.VectorSubcoreMesh(core_axis_name="c", subcore_axis_name="s")

    buf_types = []
    for _ in range(NBUF):
        buf_types += [
            pltpu.VMEM((K,), jnp.int32),
            pltpu.VMEM((K,), jnp.int32),
            pltpu.VMEM((K, D), jnp.float32),
            pltpu.VMEM((K,), jnp.float32),
            pltpu.VMEM((K,), jnp.float32),
            pltpu.VMEM((K, 16), jnp.float32),
            pltpu.SemaphoreType.DMA,
            pltpu.SemaphoreType.DMA,
        ]

    @functools.partial(
        pl.kernel,
        mesh=mesh,
        compiler_params=pltpu.CompilerParams(
            use_tc_tiling_on_sc=False, needs_layout_passes=False),
        out_type=(jax.ShapeDtypeStruct((NC, Np, D), jnp.float32),
                  jax.ShapeDtypeStruct((NC, Np, 16), jnp.float32)),
        scratch_types=buf_types + [
            pltpu.VMEM_SHARED((Np, D), jnp.float32),
            pltpu.VMEM_SHARED((Np, 16), jnp.float32),
        ],
    )
    def edge_kernel(src_h, dst_h, feat_h, el_h, er_h, outf_h, outw_h, *scr):
        bufs = [scr[i * 8:(i + 1) * 8] for i in range(NBUF)]
        accf, accw = scr[NBUF * 8], scr[NBUF * 8 + 1]
        cid = lax.axis_index("c")
        sid = lax.axis_index("s")
        tid = cid * NS + sid
        zeros16 = jnp.zeros((16,), jnp.float32)

        rows0, wbr0 = bufs[0][2], bufs[0][5]

        def zrow(i, carry):
            for j in range(D // 16):
                rows0[i, pl.ds(j * 16, 16)] = zeros16
            wbr0[i, pl.ds(0, 16)] = zeros16
            return carry

        lax.fori_loop(0, K, zrow, 0)
        for q in range(RPT // K):
            pltpu.sync_copy(rows0, accf.at[pl.ds(sid * RPT + q * K, K)])
            pltpu.sync_copy(wbr0, accw.at[pl.ds(sid * RPT + q * K, K)])
        plsc.subcore_barrier()

        def issue_gathers(b, ci):
            sidx, didx, rows, els, erd, wbr, gsem, ssem = bufs[b]
            base = tid * EPT + ci * K
            pltpu.async_copy(src_h.at[pl.ds(base, K)], sidx, gsem)
            pltpu.async_copy(dst_h.at[pl.ds(base, K)], didx, gsem)

        def wait_idx_and_fetch(b):
            sidx, didx, rows, els, erd, wbr, gsem, ssem = bufs[b]
            pltpu.make_async_copy(src_h.at[pl.ds(0, K)], sidx, gsem).wait()
            pltpu.make_async_copy(dst_h.at[pl.ds(0, K)], didx, gsem).wait()
            pltpu.async_copy(feat_h.at[sidx], rows, gsem)
            pltpu.async_copy(el_h.at[sidx], els, gsem)
            pltpu.async_copy(er_h.at[didx], erd, gsem)

        def wait_gathers(b):
            sidx, didx, rows, els, erd, wbr, gsem, ssem = bufs[b]
            pltpu.make_async_copy(feat_h.at[sidx], rows, gsem).wait()
            pltpu.make_async_copy(el_h.at[sidx], els, gsem).wait()
            pltpu.make_async_copy(er_h.at[didx], erd, gsem).wait()

        def compute(b):
            sidx, didx, rows, els, erd, wbr, gsem, ssem = bufs[b]
            for g in range(K // 16):
                e = els[pl.ds(g * 16, 16)] + erd[pl.ds(g * 16, 16)]
                e = jnp.where(e >= 0.0, e, e * 0.2)
                els[pl.ds(g * 16, 16)] = jnp.exp(e)

            @plsc.parallel_loop(0, K, 1, unroll=4)
            def edge(k):
                wv = plsc.load_gather(els, [jnp.full((16,), k, jnp.int32)])
                for j in range(D // 16):
                    rows[k, pl.ds(j * 16, 16)] = rows[k, pl.ds(j * 16, 16)] * wv
                wbr[k, pl.ds(0, 16)] = wv

        def issue_scatter(b):
            sidx, didx, rows, els, erd, wbr, gsem, ssem = bufs[b]
            pltpu.async_copy(rows, accf.at[didx], ssem, add=True)
            pltpu.async_copy(wbr, accw.at[didx], ssem, add=True)

        def wait_scatter(b):
            sidx, didx, rows, els, erd, wbr, gsem, ssem = bufs[b]
            pltpu.make_async_copy(rows, accf.at[didx], ssem).wait()
            pltpu.make_async_copy(wbr, accw.at[didx], ssem).wait()

        issue_gathers(0, 0)
        issue_gathers(1, 1)
        wait_idx_and_fetch(0)
        wait_idx_and_fetch(1)

        def body(ci, carry):
            for r in range(NBUF):
                @pl.when(ci % NBUF == r)
                def _():
                    wait_gathers(r)
                    compute(r)
                    issue_scatter(r)
                    nb = (r + 2) % NBUF

                    @pl.when(ci + 2 < NCHUNK)
                    def _():
                        @pl.when(ci >= 1)
                        def _():
                            wait_scatter(nb)

                        issue_gathers(nb, ci + 2)
                        wait_idx_and_fetch(nb)
            return carry

        assert NCHUNK >= NBUF
        lax.fori_loop(0, NCHUNK, body, 0)
        for r in range(NBUF):
            wait_scatter(r)
        plsc.subcore_barrier()

        pltpu.sync_copy(accf.at[pl.ds(sid * RPT, RPT)],
                        outf_h.at[cid, pl.ds(sid * RPT, RPT)])
        pltpu.sync_copy(accw.at[pl.ds(sid * RPT, RPT)],
                        outw_h.at[cid, pl.ds(sid * RPT, RPT)])

    return edge_kernel


def kernel(x, edge_index, W1, al1, ar1, b1, gamma, beta, W2, al2, ar2, b2):
    N, Din = x.shape
    E = edge_index.shape[1]
    Dh = W1.shape[1]
    Do = W2.shape[1]

    src = edge_index[0].astype(jnp.int32)
    dst = edge_index[1].astype(jnp.int32)

    Np = ((N + 2047) // 2048) * 2048

    feat1, el1, er1 = _pre_stage(x, W1, al1, ar1)

    acc1f, acc1w = _make_edge_kernel(N, Np, E, Dh, 80)(src, dst, feat1, el1, er1)

    feat2, el2, er2 = _mid_stage(acc1f, acc1w, N, b1.reshape(1, Dh),
                                 gamma.reshape(1, Dh), beta.reshape(1, Dh),
                                 W2, al2, ar2)

    acc2f, acc2w = _make_edge_kernel(N, Np, E, Do, 80)(src, dst, feat2, el2, er2)

    return _final_stage(acc2f, acc2w, N, b2.reshape(1, Do))


# speedup vs baseline: 53.5198x; 1.0360x over previous
import functools

import jax
import jax.numpy as jnp
from jax import lax
from jax.experimental import pallas as pl
from jax.experimental.pallas import tpu as pltpu
from jax.experimental.pallas import tpu_sc as plsc


def _pre_stage(x, W, al, ar):
    N = x.shape[0]
    Dh = W.shape[1]

    def body(x_ref, w_ref, al_ref, ar_ref, feat_ref, el_ref, er_ref):
        feat = jnp.dot(x_ref[...], w_ref[...], preferred_element_type=jnp.float32)
        feat_ref[...] = feat
        el_ref[...] = jnp.sum(feat * al_ref[...], axis=1)
        er_ref[...] = jnp.sum(feat * ar_ref[...], axis=1)

    return pl.pallas_call(
        body,
        out_shape=[
            jax.ShapeDtypeStruct((N, Dh), jnp.float32),
            jax.ShapeDtypeStruct((N,), jnp.float32),
            jax.ShapeDtypeStruct((N,), jnp.float32),
        ],
    )(x, W, al, ar)


def _bn(h, gamma, beta, eps=1e-5):
    m = jnp.mean(h, axis=0, keepdims=True)
    v = jnp.mean((h - m) ** 2, axis=0, keepdims=True)
    return (h - m) / jnp.sqrt(v + eps) * gamma + beta


def _mid_stage(accf, accw, N, b1, gamma, beta, W2, al2, ar2):
    D = b1.shape[1]
    Do = W2.shape[1]

    def body(accf_ref, accw_ref, b_ref, g_ref, be_ref, w_ref, al_ref, ar_ref,
             feat_ref, el_ref, er_ref):
        a = accf_ref[0, :N] + accf_ref[1, :N]
        denom = accw_ref[0, :N, 0:1] + accw_ref[1, :N, 0:1]
        denom = jnp.where(denom == 0.0, 1.0, denom)
        rst = a / denom + b_ref[...]
        h = jnp.where(rst > 0, rst, jnp.exp(jnp.minimum(rst, 0.0)) - 1.0)
        h = _bn(h, g_ref[...], be_ref[...])
        h = _bn(h, g_ref[...], be_ref[...])
        feat = jnp.dot(h, w_ref[...], preferred_element_type=jnp.float32)
        feat_ref[...] = feat
        el_ref[...] = jnp.sum(feat * al_ref[...], axis=1)
        er_ref[...] = jnp.sum(feat * ar_ref[...], axis=1)

    return pl.pallas_call(
        body,
        out_shape=[
            jax.ShapeDtypeStruct((N, Do), jnp.float32),
            jax.ShapeDtypeStruct((N,), jnp.float32),
            jax.ShapeDtypeStruct((N,), jnp.float32),
        ],
    )(accf, accw, b1, gamma, beta, W2, al2, ar2)


def _final_stage(accf, accw, N, b2):
    D = b2.shape[1]

    def body(accf_ref, accw_ref, b_ref, out_ref):
        a = accf_ref[0, :N] + accf_ref[1, :N]
        denom = accw_ref[0, :N, 0:1] + accw_ref[1, :N, 0:1]
        denom = jnp.where(denom == 0.0, 1.0, denom)
        out_ref[...] = a / denom + b_ref[...]

    return pl.pallas_call(
        body,
        out_shape=jax.ShapeDtypeStruct((N, D), jnp.float32),
    )(accf, accw, b2)


@functools.cache
def _make_edge_kernel(N, Np, E, D, K):
    NBUF = 3
    info = plsc.get_sparse_core_info()
    NC, NS = info.num_cores, info.num_subcores
    NW = NC * NS
    EPT = E // NW
    assert E % NW == 0 and EPT % K == 0 and K % 16 == 0
    NCHUNK = EPT // K
    RPT = Np // NS
    assert Np % NS == 0 and RPT % 8 == 0 and RPT % K == 0
    mesh = plsc.VectorSubcoreMesh(core_axis_name="c", subcore_axis_name="s")

    buf_types = []
    for _ in range(NBUF):
        buf_types += [
            pltpu.VMEM((K,), jnp.int32),
            pltpu.VMEM((K,), jnp.int32),
            pltpu.VMEM((K, D), jnp.float32),
            pltpu.VMEM((K,), jnp.float32),
            pltpu.VMEM((K,), jnp.float32),
            pltpu.VMEM((K, 16), jnp.float32),
            pltpu.SemaphoreType.DMA,
            pltpu.SemaphoreType.DMA,
        ]

    @functools.partial(
        pl.kernel,
        mesh=mesh,
        compiler_params=pltpu.CompilerParams(
            use_tc_tiling_on_sc=False, needs_layout_passes=False),
        out_type=(jax.ShapeDtypeStruct((NC, Np, D), jnp.float32),
                  jax.ShapeDtypeStruct((NC, Np, 16), jnp.float32)),
        scratch_types=buf_types + [
            pltpu.VMEM_SHARED((Np, D), jnp.float32),
            pltpu.VMEM_SHARED((Np, 16), jnp.float32),
        ],
    )
    def edge_kernel(ei_h, feat_h, el_h, er_h, outf_h, outw_h, *scr):
        bufs = [scr[i * 8:(i + 1) * 8] for i in range(NBUF)]
        accf, accw = scr[NBUF * 8], scr[NBUF * 8 + 1]
        cid = lax.axis_index("c")
        sid = lax.axis_index("s")
        tid = cid * NS + sid
        zeros16 = jnp.zeros((16,), jnp.float32)

        rows0, wbr0 = bufs[0][2], bufs[0][5]

        def zrow(i, carry):
            for j in range(D // 16):
                rows0[i, pl.ds(j * 16, 16)] = zeros16
            wbr0[i, pl.ds(0, 16)] = zeros16
            return carry

        lax.fori_loop(0, K, zrow, 0)
        for q in range(RPT // K):
            pltpu.sync_copy(rows0, accf.at[pl.ds(sid * RPT + q * K, K)])
            pltpu.sync_copy(wbr0, accw.at[pl.ds(sid * RPT + q * K, K)])
        plsc.subcore_barrier()

        def issue_gathers(b, ci):
            sidx, didx, rows, els, erd, wbr, gsem, ssem = bufs[b]
            base = tid * EPT + ci * K
            pltpu.async_copy(ei_h.at[0, pl.ds(base, K)], sidx, gsem)
            pltpu.async_copy(ei_h.at[1, pl.ds(base, K)], didx, gsem)

        def wait_idx_and_fetch(b):
            sidx, didx, rows, els, erd, wbr, gsem, ssem = bufs[b]
            pltpu.make_async_copy(ei_h.at[0, pl.ds(0, K)], sidx, gsem).wait()
            pltpu.make_async_copy(ei_h.at[1, pl.ds(0, K)], didx, gsem).wait()
            pltpu.async_copy(feat_h.at[sidx], rows, gsem)
            pltpu.async_copy(el_h.at[sidx], els, gsem)
            pltpu.async_copy(er_h.at[didx], erd, gsem)

        def wait_gathers(b):
            sidx, didx, rows, els, erd, wbr, gsem, ssem = bufs[b]
            pltpu.make_async_copy(feat_h.at[sidx], rows, gsem).wait()
            pltpu.make_async_copy(el_h.at[sidx], els, gsem).wait()
            pltpu.make_async_copy(er_h.at[didx], erd, gsem).wait()

        def compute(b):
            sidx, didx, rows, els, erd, wbr, gsem, ssem = bufs[b]
            for g in range(K // 16):
                e = els[pl.ds(g * 16, 16)] + erd[pl.ds(g * 16, 16)]
                e = jnp.where(e >= 0.0, e, e * 0.2)
                els[pl.ds(g * 16, 16)] = jnp.exp(e)

            @plsc.parallel_loop(0, K, 1, unroll=4)
            def edge(k):
                wv = plsc.load_gather(els, [jnp.full((16,), k, jnp.int32)])
                for j in range(D // 16):
                    rows[k, pl.ds(j * 16, 16)] = rows[k, pl.ds(j * 16, 16)] * wv
                wbr[k, pl.ds(0, 16)] = wv

        def issue_scatter(b):
            sidx, didx, rows, els, erd, wbr, gsem, ssem = bufs[b]
            pltpu.async_copy(rows, accf.at[didx], ssem, add=True)
            pltpu.async_copy(wbr, accw.at[didx], ssem, add=True)

        def wait_scatter(b):
            sidx, didx, rows, els, erd, wbr, gsem, ssem = bufs[b]
            pltpu.make_async_copy(rows, accf.at[didx], ssem).wait()
            pltpu.make_async_copy(wbr, accw.at[didx], ssem).wait()

        issue_gathers(0, 0)
        issue_gathers(1, 1)
        wait_idx_and_fetch(0)
        wait_idx_and_fetch(1)

        def body(ci, carry):
            for r in range(NBUF):
                @pl.when(ci % NBUF == r)
                def _():
                    wait_gathers(r)
                    compute(r)
                    issue_scatter(r)
                    nb = (r + 2) % NBUF

                    @pl.when(ci + 2 < NCHUNK)
                    def _():
                        @pl.when(ci >= 1)
                        def _():
                            wait_scatter(nb)

                        issue_gathers(nb, ci + 2)
                        wait_idx_and_fetch(nb)
            return carry

        assert NCHUNK >= NBUF
        lax.fori_loop(0, NCHUNK, body, 0)
        for r in range(NBUF):
            wait_scatter(r)
        plsc.subcore_barrier()

        pltpu.sync_copy(accf.at[pl.ds(sid * RPT, RPT)],
                        outf_h.at[cid, pl.ds(sid * RPT, RPT)])
        pltpu.sync_copy(accw.at[pl.ds(sid * RPT, RPT)],
                        outw_h.at[cid, pl.ds(sid * RPT, RPT)])

    return edge_kernel


def kernel(x, edge_index, W1, al1, ar1, b1, gamma, beta, W2, al2, ar2, b2):
    N, Din = x.shape
    E = edge_index.shape[1]
    Dh = W1.shape[1]
    Do = W2.shape[1]

    ei = edge_index.astype(jnp.int32)

    Np = ((N + 2047) // 2048) * 2048

    feat1, el1, er1 = _pre_stage(x, W1, al1, ar1)

    acc1f, acc1w = _make_edge_kernel(N, Np, E, Dh, 80)(ei, feat1, el1, er1)

    feat2, el2, er2 = _mid_stage(acc1f, acc1w, N, b1.reshape(1, Dh),
                                 gamma.reshape(1, Dh), beta.reshape(1, Dh),
                                 W2, al2, ar2)

    acc2f, acc2w = _make_edge_kernel(N, Np, E, Do, 80)(ei, feat2, el2, er2)

    return _final_stage(acc2f, acc2w, N, b2.reshape(1, Do))


# speedup vs baseline: 54.4138x; 1.0167x over previous
import functools

import jax
import jax.numpy as jnp
from jax import lax
from jax.experimental import pallas as pl
from jax.experimental.pallas import tpu as pltpu
from jax.experimental.pallas import tpu_sc as plsc


def _pre_stage(x, W, al, ar):
    N = x.shape[0]
    Dh = W.shape[1]

    def body(x_ref, w_ref, al_ref, ar_ref, feat_ref, el_ref, er_ref):
        feat = jnp.dot(x_ref[...], w_ref[...], preferred_element_type=jnp.float32)
        feat_ref[...] = feat
        el_ref[...] = jnp.sum(feat * al_ref[...], axis=1)
        er_ref[...] = jnp.sum(feat * ar_ref[...], axis=1)

    return pl.pallas_call(
        body,
        out_shape=[
            jax.ShapeDtypeStruct((N, Dh), jnp.float32),
            jax.ShapeDtypeStruct((N,), jnp.float32),
            jax.ShapeDtypeStruct((N,), jnp.float32),
        ],
    )(x, W, al, ar)


def _bn(h, ones_row, gamma, beta, eps=1e-5):
    n = h.shape[0]
    s1 = jnp.dot(ones_row, h, preferred_element_type=jnp.float32)
    s2 = jnp.dot(ones_row, h * h, preferred_element_type=jnp.float32)
    m = s1 / n
    v = jnp.maximum(s2 / n - m * m, 0.0)
    return (h - m) / jnp.sqrt(v + eps) * gamma + beta


def _mid_stage(accf, accw, N, b1, gamma, beta, W2, al2, ar2):
    D = b1.shape[1]
    Do = W2.shape[1]

    def body(accf_ref, accw_ref, b_ref, g_ref, be_ref, w_ref, al_ref, ar_ref,
             feat_ref, el_ref, er_ref):
        a = accf_ref[0, :N] + accf_ref[1, :N]
        denom = accw_ref[0, :N, 0:1] + accw_ref[1, :N, 0:1]
        denom = jnp.where(denom == 0.0, 1.0, denom)
        rst = a / denom + b_ref[...]
        h = jnp.where(rst > 0, rst, jnp.exp(jnp.minimum(rst, 0.0)) - 1.0)
        ones_row = jnp.ones((1, N), jnp.float32)
        h = _bn(h, ones_row, g_ref[...], be_ref[...])
        h = _bn(h, ones_row, g_ref[...], be_ref[...])
        feat = jnp.dot(h, w_ref[...], preferred_element_type=jnp.float32)
        feat_ref[...] = feat
        el_ref[...] = jnp.sum(feat * al_ref[...], axis=1)
        er_ref[...] = jnp.sum(feat * ar_ref[...], axis=1)

    return pl.pallas_call(
        body,
        out_shape=[
            jax.ShapeDtypeStruct((N, Do), jnp.float32),
            jax.ShapeDtypeStruct((N,), jnp.float32),
            jax.ShapeDtypeStruct((N,), jnp.float32),
        ],
    )(accf, accw, b1, gamma, beta, W2, al2, ar2)


def _final_stage(accf, accw, N, b2):
    D = b2.shape[1]

    def body(accf_ref, accw_ref, b_ref, out_ref):
        a = accf_ref[0, :N] + accf_ref[1, :N]
        denom = accw_ref[0, :N, 0:1] + accw_ref[1, :N, 0:1]
        denom = jnp.where(denom == 0.0, 1.0, denom)
        out_ref[...] = a / denom + b_ref[...]

    return pl.pallas_call(
        body,
        out_shape=jax.ShapeDtypeStruct((N, D), jnp.float32),
    )(accf, accw, b2)


@functools.cache
def _make_edge_kernel(N, Np, E, D, K):
    NBUF = 3
    info = plsc.get_sparse_core_info()
    NC, NS = info.num_cores, info.num_subcores
    NW = NC * NS
    EPT = E // NW
    assert E % NW == 0 and EPT % K == 0 and K % 16 == 0
    NCHUNK = EPT // K
    RPT = Np // NS
    assert Np % NS == 0 and RPT % 8 == 0 and RPT % K == 0
    mesh = plsc.VectorSubcoreMesh(core_axis_name="c", subcore_axis_name="s")

    buf_types = []
    for _ in range(NBUF):
        buf_types += [
            pltpu.VMEM((K,), jnp.int32),
            pltpu.VMEM((K,), jnp.int32),
            pltpu.VMEM((K, D), jnp.float32),
            pltpu.VMEM((K,), jnp.float32),
            pltpu.VMEM((K,), jnp.float32),
            pltpu.VMEM((K, 16), jnp.float32),
            pltpu.SemaphoreType.DMA,
            pltpu.SemaphoreType.DMA,
        ]

    @functools.partial(
        pl.kernel,
        mesh=mesh,
        compiler_params=pltpu.CompilerParams(
            use_tc_tiling_on_sc=False, needs_layout_passes=False),
        out_type=(jax.ShapeDtypeStruct((NC, Np, D), jnp.float32),
                  jax.ShapeDtypeStruct((NC, Np, 16), jnp.float32)),
        scratch_types=buf_types + [
            pltpu.VMEM_SHARED((Np, D), jnp.float32),
            pltpu.VMEM_SHARED((Np, 16), jnp.float32),
        ],
    )
    def edge_kernel(ei_h, feat_h, el_h, er_h, outf_h, outw_h, *scr):
        bufs = [scr[i * 8:(i + 1) * 8] for i in range(NBUF)]
        accf, accw = scr[NBUF * 8], scr[NBUF * 8 + 1]
        cid = lax.axis_index("c")
        sid = lax.axis_index("s")
        tid = cid * NS + sid
        zeros16 = jnp.zeros((16,), jnp.float32)

        rows0, wbr0 = bufs[0][2], bufs[0][5]

        def zrow(i, carry):
            for j in range(D // 16):
                rows0[i, pl.ds(j * 16, 16)] = zeros16
            wbr0[i, pl.ds(0, 16)] = zeros16
            return carry

        lax.fori_loop(0, K, zrow, 0)
        for q in range(RPT // K):
            pltpu.sync_copy(rows0, accf.at[pl.ds(sid * RPT + q * K, K)])
            pltpu.sync_copy(wbr0, accw.at[pl.ds(sid * RPT + q * K, K)])
        plsc.subcore_barrier()

        def issue_gathers(b, ci):
            sidx, didx, rows, els, erd, wbr, gsem, ssem = bufs[b]
            base = tid * EPT + ci * K
            pltpu.async_copy(ei_h.at[0, pl.ds(base, K)], sidx, gsem)
            pltpu.async_copy(ei_h.at[1, pl.ds(base, K)], didx, gsem)

        def wait_idx_and_fetch(b):
            sidx, didx, rows, els, erd, wbr, gsem, ssem = bufs[b]
            pltpu.make_async_copy(ei_h.at[0, pl.ds(0, K)], sidx, gsem).wait()
            pltpu.make_async_copy(ei_h.at[1, pl.ds(0, K)], didx, gsem).wait()
            pltpu.async_copy(feat_h.at[sidx], rows, gsem)
            pltpu.async_copy(el_h.at[sidx], els, gsem)
            pltpu.async_copy(er_h.at[didx], erd, gsem)

        def wait_gathers(b):
            sidx, didx, rows, els, erd, wbr, gsem, ssem = bufs[b]
            pltpu.make_async_copy(feat_h.at[sidx], rows, gsem).wait()
            pltpu.make_async_copy(el_h.at[sidx], els, gsem).wait()
            pltpu.make_async_copy(er_h.at[didx], erd, gsem).wait()

        def compute(b):
            sidx, didx, rows, els, erd, wbr, gsem, ssem = bufs[b]
            for g in range(K // 16):
                e = els[pl.ds(g * 16, 16)] + erd[pl.ds(g * 16, 16)]
                e = jnp.where(e >= 0.0, e, e * 0.2)
                els[pl.ds(g * 16, 16)] = jnp.exp(e)

            @plsc.parallel_loop(0, K, 1, unroll=4)
            def edge(k):
                wv = plsc.load_gather(els, [jnp.full((16,), k, jnp.int32)])
                for j in range(D // 16):
                    rows[k, pl.ds(j * 16, 16)] = rows[k, pl.ds(j * 16, 16)] * wv
                wbr[k, pl.ds(0, 16)] = wv

        def issue_scatter(b):
            sidx, didx, rows, els, erd, wbr, gsem, ssem = bufs[b]
            pltpu.async_copy(rows, accf.at[didx], ssem, add=True)
            pltpu.async_copy(wbr, accw.at[didx], ssem, add=True)

        def wait_scatter(b):
            sidx, didx, rows, els, erd, wbr, gsem, ssem = bufs[b]
            pltpu.make_async_copy(rows, accf.at[didx], ssem).wait()
            pltpu.make_async_copy(wbr, accw.at[didx], ssem).wait()

        issue_gathers(0, 0)
        issue_gathers(1, 1)
        wait_idx_and_fetch(0)
        wait_idx_and_fetch(1)

        def body(ci, carry):
            for r in range(NBUF):
                @pl.when(ci % NBUF == r)
                def _():
                    wait_gathers(r)
                    compute(r)
                    issue_scatter(r)
                    nb = (r + 2) % NBUF

                    @pl.when(ci + 2 < NCHUNK)
                    def _():
                        @pl.when(ci >= 1)
                        def _():
                            wait_scatter(nb)

                        issue_gathers(nb, ci + 2)
                        wait_idx_and_fetch(nb)
            return carry

        assert NCHUNK >= NBUF
        lax.fori_loop(0, NCHUNK, body, 0)
        for r in range(NBUF):
            wait_scatter(r)
        plsc.subcore_barrier()

        pltpu.sync_copy(accf.at[pl.ds(sid * RPT, RPT)],
                        outf_h.at[cid, pl.ds(sid * RPT, RPT)])
        pltpu.sync_copy(accw.at[pl.ds(sid * RPT, RPT)],
                        outw_h.at[cid, pl.ds(sid * RPT, RPT)])

    return edge_kernel


def kernel(x, edge_index, W1, al1, ar1, b1, gamma, beta, W2, al2, ar2, b2):
    N, Din = x.shape
    E = edge_index.shape[1]
    Dh = W1.shape[1]
    Do = W2.shape[1]

    ei = edge_index.astype(jnp.int32)

    Np = ((N + 2047) // 2048) * 2048

    feat1, el1, er1 = _pre_stage(x, W1, al1, ar1)

    acc1f, acc1w = _make_edge_kernel(N, Np, E, Dh, 80)(ei, feat1, el1, er1)

    feat2, el2, er2 = _mid_stage(acc1f, acc1w, N, b1.reshape(1, Dh),
                                 gamma.reshape(1, Dh), beta.reshape(1, Dh),
                                 W2, al2, ar2)

    acc2f, acc2w = _make_edge_kernel(N, Np, E, Do, 80)(ei, feat2, el2, er2)

    return _final_stage(acc2f, acc2w, N, b2.reshape(1, Do))
